# Initial kernel scaffold; baseline (speedup 1.0000x reference)
import functools

import jax
import jax.numpy as jnp
from jax import lax
from jax.experimental import pallas as pl
from jax.experimental.pallas import tpu as pltpu
from jax.experimental.pallas import tpu_sc as plsc

N_U = 5000
N_P = 5000
N = N_U + N_P
E = 320000
H = 128
EPS = 1e-5

NC = 2
NS = 16
NW = NC * NS
E2 = 2 * E
BE = 128
EP = 655360
NB = EP // (NW * BE)
ROWS_OUT = N // NS
AGG_ROWS = N + NS
ZROWS = AGG_ROWS // NS


def _seg_sum_call(feat, src_r, dst_r, zeros_feat, zeros_cnt, ones_cnt,
                  with_count):
  out_type = [jax.ShapeDtypeStruct((NC, N, H), jnp.float32)]
  scratch = [
      pltpu.VMEM((NB, BE), jnp.int32),
      pltpu.VMEM((NB, BE), jnp.int32),
      pltpu.VMEM((BE, H), jnp.float32),
      pltpu.VMEM_SHARED((AGG_ROWS, H), jnp.float32),
      pltpu.SemaphoreType.DMA,
  ]
  if with_count:
    out_type.append(jax.ShapeDtypeStruct((NC, N, 16), jnp.float32))
    scratch += [
        pltpu.VMEM((BE, 16), jnp.float32),
        pltpu.VMEM_SHARED((AGG_ROWS, 16), jnp.float32),
    ]

  mesh = plsc.VectorSubcoreMesh(core_axis_name="c", subcore_axis_name="s")

  def body(*refs):
    if with_count:
      (feat_h, src_h, dst_h, zf_h, zc_h, ones_h,
       out_agg, out_cnt, sidx, didx, rows, agg, sem, ones, cagg) = refs
    else:
      (feat_h, src_h, dst_h, zf_h,
       out_agg, sidx, didx, rows, agg, sem) = refs
    c = lax.axis_index("c")
    s = lax.axis_index("s")
    w = c * NS + s

    pltpu.sync_copy(zf_h.at[pl.ds(s * ZROWS, ZROWS)],
                    agg.at[pl.ds(s * ZROWS, ZROWS)])
    pltpu.sync_copy(src_h.at[w], sidx)
    pltpu.sync_copy(dst_h.at[w], didx)
    if with_count:
      pltpu.sync_copy(zc_h.at[pl.ds(s * ZROWS, ZROWS)],
                      cagg.at[pl.ds(s * ZROWS, ZROWS)])
      pltpu.sync_copy(ones_h, ones)
    plsc.subcore_barrier()

    def step(j, carry):
      pltpu.async_copy(feat_h.at[sidx.at[j]], rows, sem).wait()
      pltpu.sync_copy(rows, agg.at[didx.at[j]], add=True)
      if with_count:
        pltpu.sync_copy(ones, cagg.at[didx.at[j]], add=True)
      return carry

    lax.fori_loop(0, NB, step, 0)
    plsc.subcore_barrier()

    pltpu.sync_copy(agg.at[pl.ds(s * ROWS_OUT, ROWS_OUT)],
                    out_agg.at[c, pl.ds(s * ROWS_OUT, ROWS_OUT)])
    if with_count:
      pltpu.sync_copy(cagg.at[pl.ds(s * ROWS_OUT, ROWS_OUT)],
                      out_cnt.at[c, pl.ds(s * ROWS_OUT, ROWS_OUT)])

  fn = pl.kernel(body, out_type=out_type, mesh=mesh, scratch_types=scratch)
  if with_count:
    return fn(feat, src_r, dst_r, zeros_feat, zeros_cnt, ones_cnt)
  return fn(feat, src_r, dst_r, zeros_feat)


def _enc_body(x_ref, wt_ref, b_ref, o_ref):
  o_ref[...] = jnp.maximum(
      jnp.dot(x_ref[0], wt_ref[0], preferred_element_type=jnp.float32)
      + b_ref[0], 0.0)[None]


def _encode(xs, wts, bs):
  return pl.pallas_call(
      _enc_body,
      grid=(2,),
      in_specs=[
          pl.BlockSpec((1, N_U, H), lambda g: (g, 0, 0)),
          pl.BlockSpec((1, H, H), lambda g: (g, 0, 0)),
          pl.BlockSpec((1, 1, H), lambda g: (g, 0, 0)),
      ],
      out_specs=pl.BlockSpec((1, N_U, H), lambda g: (g, 0, 0)),
      out_shape=jax.ShapeDtypeStruct((2, N_U, H), jnp.float32),
  )(xs, wts, bs)


def _mid_body(aggA, aggB, cA, cB, x_ref, wlt, bl, wrt, gs, be, o_ref):
  cnt = cA[:, :1] + cB[:, :1]
  mean = (aggA[...] + aggB[...]) / jnp.maximum(cnt, 1.0)
  h = (jnp.dot(mean, wlt[...], preferred_element_type=jnp.float32) + bl[...]
       + jnp.dot(x_ref[...], wrt[...], preferred_element_type=jnp.float32))
  o_ref[...] = jnp.maximum(h * gs[...] + be[...], 0.0)


def _mid_layer(aggA, aggB, cA, cB, x, wlt, bl, wrt, gs, be):
  n = x.shape[0]
  args = (aggA, aggB, cA, cB, x, wlt, bl, wrt, gs, be)
  specs = [pl.BlockSpec(a.shape, lambda *_: tuple(0 for _ in a.shape))
           for a in args]
  return pl.pallas_call(
      _mid_body,
      in_specs=specs,
      out_specs=pl.BlockSpec((n, H), lambda *_: (0, 0)),
      out_shape=jax.ShapeDtypeStruct((n, H), jnp.float32),
  )(*args)


def _fin_body(aggA, aggB, cA, cB, h1, x0, wlt, bl, wrt, gs, be, wot, bo,
              o_ref):
  cnt = cA[:, :1] + cB[:, :1]
  mean = (aggA[...] + aggB[...]) / jnp.maximum(cnt, 1.0)
  h = (jnp.dot(mean, wlt[...], preferred_element_type=jnp.float32) + bl[...]
       + jnp.dot(h1[...], wrt[...], preferred_element_type=jnp.float32))
  h = jnp.maximum(h * gs[...] + be[...], 0.0) + x0[...]
  o_ref[...] = jnp.dot(h, wot[...], preferred_element_type=jnp.float32) + bo[...]


def _fin_layer(aggA, aggB, cA, cB, h1, x0, wlt, bl, wrt, gs, be, wot, bo):
  args = (aggA, aggB, cA, cB, h1, x0, wlt, bl, wrt, gs, be, wot, bo)
  specs = [pl.BlockSpec(a.shape, lambda *_: tuple(0 for _ in a.shape))
           for a in args]
  return pl.pallas_call(
      _fin_body,
      in_specs=specs,
      out_specs=pl.BlockSpec((N_U, H), lambda *_: (0, 0)),
      out_shape=jax.ShapeDtypeStruct((N_U, H), jnp.float32),
  )(*args)


def kernel(x_u, x_p, edge_index, W_u, b_u, W_p, b_p, W1_l, b1_l, W1_r, g1,
           be1, W2_l, b2_l, W2_r, g2, be2, W_out, b_out):
  s = 1.0 / jnp.sqrt(jnp.float32(1.0 + EPS))

  src = jnp.concatenate([edge_index[0], edge_index[1]]).astype(jnp.int32)
  dst = jnp.concatenate([edge_index[1], edge_index[0]]).astype(jnp.int32)
  pad = EP - E2
  src_r = jnp.concatenate([src, jnp.zeros((pad,), jnp.int32)]
                          ).reshape(NW, NB, BE)
  dst_r = jnp.concatenate([dst, jnp.full((pad,), N, jnp.int32)]
                          ).reshape(NW, NB, BE)
  zeros_feat = jnp.zeros((AGG_ROWS, H), jnp.float32)
  zeros_cnt = jnp.zeros((AGG_ROWS, 16), jnp.float32)
  ones_cnt = jnp.zeros((BE, 16), jnp.float32).at[:, 0].set(1.0)

  xs = jnp.stack([x_u, x_p])
  wts = jnp.stack([W_u.T, W_p.T])
  bs = jnp.stack([b_u[None], b_p[None]])
  x0 = _encode(xs, wts, bs).reshape(N, H)

  agg1, cnt1 = _seg_sum_call(x0, src_r, dst_r, zeros_feat, zeros_cnt,
                             ones_cnt, with_count=True)
  h1 = _mid_layer(agg1[0], agg1[1], cnt1[0], cnt1[1], x0,
                  W1_l.T, b1_l[None], W1_r.T, (g1 * s)[None], be1[None])

  agg2 = _seg_sum_call(h1, src_r, dst_r, zeros_feat, zeros_cnt,
                       ones_cnt, with_count=False)
  if isinstance(agg2, (tuple, list)):
    agg2 = agg2[0]
  wot = jnp.zeros((H, H), jnp.float32).at[:, 0].set(W_out[0])
  bo = jnp.zeros((1, H), jnp.float32).at[0, 0].set(b_out[0])
  out_full = _fin_layer(agg2[0, :N_U], agg2[1, :N_U], cnt1[0, :N_U],
                        cnt1[1, :N_U], h1[:N_U], x0[:N_U],
                        W2_l.T, b2_l[None], W2_r.T, (g2 * s)[None],
                        be2[None], wot, bo)
  return out_full[:, :1]


# speedup vs baseline: 3.4578x; 3.4578x over previous
import functools

import jax
import jax.numpy as jnp
from jax import lax
from jax.experimental import pallas as pl
from jax.experimental.pallas import tpu as pltpu
from jax.experimental.pallas import tpu_sc as plsc

N_U = 5000
N_P = 5000
N = N_U + N_P
E = 320000
H = 128
EPS = 1e-5
HA = 144

NC = 2
NS = 16
NW = NC * NS
E2 = 2 * E
BE = 128
EP = 655360
NB = EP // (NW * BE)
CB = 16
AGG_ROWS = 10112
ZROWS = AGG_ROWS // NS
ROWS_OUT = ZROWS


def _seg_sum_call(feat, src_r, dst_r, zeros_feat, width):
  out_type = [jax.ShapeDtypeStruct((NC, AGG_ROWS, width), jnp.float32)]
  scratch = [
      pltpu.VMEM((CB, BE), jnp.int32),
      pltpu.VMEM((CB, BE), jnp.int32),
      pltpu.VMEM((BE, width), jnp.float32),
      pltpu.VMEM_SHARED((AGG_ROWS, width), jnp.float32),
  ]

  mesh = plsc.VectorSubcoreMesh(core_axis_name="c", subcore_axis_name="s",
                                num_cores=NC, num_subcores=NS)

  def body(feat_h, src_h, dst_h, zf_h, out_agg, sidx, didx, rows, agg):
    c = lax.axis_index("c")
    s = lax.axis_index("s")
    w = c * NS + s

    pltpu.sync_copy(zf_h.at[pl.ds(s * ZROWS, ZROWS)],
                    agg.at[pl.ds(s * ZROWS, ZROWS)])
    plsc.subcore_barrier()

    def chunk(o, carry):
      pltpu.sync_copy(src_h.at[w, pl.ds(o * CB, CB)], sidx)
      pltpu.sync_copy(dst_h.at[w, pl.ds(o * CB, CB)], didx)
      for j in range(CB):
        pltpu.sync_copy(feat_h.at[sidx.at[j]], rows)
        pltpu.sync_copy(rows, agg.at[didx.at[j]], add=True)
      return carry

    lax.fori_loop(0, NB // CB, chunk, 0)
    plsc.subcore_barrier()

    pltpu.sync_copy(agg.at[pl.ds(s * ROWS_OUT, ROWS_OUT)],
                    out_agg.at[c, pl.ds(s * ROWS_OUT, ROWS_OUT)])

  fn = pl.kernel(body, out_type=out_type, mesh=mesh, scratch_types=scratch)
  return fn(feat, src_r, dst_r, zeros_feat)[0]


def _count_call(dst_r, zeros_feat, ones_rows):
  out_type = [jax.ShapeDtypeStruct((NC, AGG_ROWS, H), jnp.float32)]
  scratch = [
      pltpu.VMEM((CB, BE), jnp.int32),
      pltpu.VMEM((BE, H), jnp.float32),
      pltpu.VMEM_SHARED((AGG_ROWS, H), jnp.float32),
  ]
  mesh = plsc.VectorSubcoreMesh(core_axis_name="c", subcore_axis_name="s",
                                num_cores=NC, num_subcores=NS)

  def body(dst_h, zf_h, ones_h, out_agg, didx, ones, agg):
    c = lax.axis_index("c")
    s = lax.axis_index("s")
    w = c * NS + s
    pltpu.sync_copy(zf_h.at[pl.ds(s * ZROWS, ZROWS)],
                    agg.at[pl.ds(s * ZROWS, ZROWS)])
    pltpu.sync_copy(ones_h, ones)
    plsc.subcore_barrier()

    def chunk(o, carry):
      pltpu.sync_copy(dst_h.at[w, pl.ds(o * CB, CB)], didx)
      for j in range(CB):
        pltpu.sync_copy(ones, agg.at[didx.at[j]], add=True)
      return carry

    lax.fori_loop(0, NB // CB, chunk, 0)
    plsc.subcore_barrier()
    pltpu.sync_copy(agg.at[pl.ds(s * ROWS_OUT, ROWS_OUT)],
                    out_agg.at[c, pl.ds(s * ROWS_OUT, ROWS_OUT)])

  fn = pl.kernel(body, out_type=out_type, mesh=mesh, scratch_types=scratch)
  return fn(dst_r, zeros_feat, ones_rows)[0]


def _enc_body(x_ref, wt_ref, b_ref, o_ref):
  o_ref[...] = jnp.maximum(
      jnp.dot(x_ref[0], wt_ref[0], preferred_element_type=jnp.float32)
      + b_ref[0], 0.0)[None]


def _encode(xs, wts, bs):
  return pl.pallas_call(
      _enc_body,
      grid=(2,),
      in_specs=[
          pl.BlockSpec((1, N_U, H), lambda g: (g, 0, 0)),
          pl.BlockSpec((1, H, H), lambda g: (g, 0, 0)),
          pl.BlockSpec((1, 1, H), lambda g: (g, 0, 0)),
      ],
      out_specs=pl.BlockSpec((1, N_U, H), lambda g: (g, 0, 0)),
      out_shape=jax.ShapeDtypeStruct((2, N_U, H), jnp.float32),
  )(xs, wts, bs)


def _mid_body(aggA, aggB, cA, cB, x_ref, wlt, bl, wrt, gs, be, o_ref):
  cnt = cA[:, :1] + cB[:, :1]
  mean = (aggA[...] + aggB[...]) / jnp.maximum(cnt, 1.0)
  h = (jnp.dot(mean, wlt[...], preferred_element_type=jnp.float32) + bl[...]
       + jnp.dot(x_ref[...], wrt[...], preferred_element_type=jnp.float32))
  o_ref[...] = jnp.maximum(h * gs[...] + be[...], 0.0)


def _mid_layer(aggA, aggB, cA, cB, x, wlt, bl, wrt, gs, be):
  n = x.shape[0]
  args = (aggA, aggB, cA, cB, x, wlt, bl, wrt, gs, be)
  specs = [pl.BlockSpec(a.shape, lambda *_: tuple(0 for _ in a.shape))
           for a in args]
  return pl.pallas_call(
      _mid_body,
      in_specs=specs,
      out_specs=pl.BlockSpec((n, H), lambda *_: (0, 0)),
      out_shape=jax.ShapeDtypeStruct((n, H), jnp.float32),
  )(*args)


def _fin_body(aggA, aggB, cA, cB, h1, x0, wlt, bl, wrt, gs, be, wot, bo,
              o_ref):
  cnt = cA[:, :1] + cB[:, :1]
  mean = (aggA[...] + aggB[...]) / jnp.maximum(cnt, 1.0)
  h = (jnp.dot(mean, wlt[...], preferred_element_type=jnp.float32) + bl[...]
       + jnp.dot(h1[...], wrt[...], preferred_element_type=jnp.float32))
  h = jnp.maximum(h * gs[...] + be[...], 0.0) + x0[...]
  o_ref[...] = jnp.dot(h, wot[...], preferred_element_type=jnp.float32) + bo[...]


def _fin_layer(aggA, aggB, cA, cB, h1, x0, wlt, bl, wrt, gs, be, wot, bo):
  args = (aggA, aggB, cA, cB, h1, x0, wlt, bl, wrt, gs, be, wot, bo)
  specs = [pl.BlockSpec(a.shape, lambda *_: tuple(0 for _ in a.shape))
           for a in args]
  return pl.pallas_call(
      _fin_body,
      in_specs=specs,
      out_specs=pl.BlockSpec((N_U, H), lambda *_: (0, 0)),
      out_shape=jax.ShapeDtypeStruct((N_U, H), jnp.float32),
  )(*args)


def kernel(x_u, x_p, edge_index, W_u, b_u, W_p, b_p, W1_l, b1_l, W1_r, g1,
           be1, W2_l, b2_l, W2_r, g2, be2, W_out, b_out):
  s = 1.0 / jnp.sqrt(jnp.float32(1.0 + EPS))

  src = jnp.concatenate([edge_index[0], edge_index[1]]).astype(jnp.int32)
  dst = jnp.concatenate([edge_index[1], edge_index[0]]).astype(jnp.int32)
  pad = EP - E2
  src_r = jnp.concatenate([src, jnp.zeros((pad,), jnp.int32)]
                          ).reshape(NW, NB, BE)
  dst_r = jnp.concatenate([dst, jnp.full((pad,), N, jnp.int32)]
                          ).reshape(NW, NB, BE)
  zeros_feat = jnp.zeros((AGG_ROWS, H), jnp.float32)
  ones_rows = jnp.zeros((BE, H), jnp.float32).at[:, 0].set(1.0)

  xs = jnp.stack([x_u, x_p])
  wts = jnp.stack([W_u.T, W_p.T])
  bs = jnp.stack([b_u[None], b_p[None]])
  x0 = _encode(xs, wts, bs).reshape(N, H)

  cnt = _count_call(dst_r, zeros_feat, ones_rows)
  tok = (cnt[0, 0, 1] * 0.0).astype(jnp.int32)
  agg1 = _seg_sum_call(x0, src_r + tok, dst_r, zeros_feat, width=H)
  h1 = _mid_layer(agg1[0, :N], agg1[1, :N],
                  cnt[0, :N, :16], cnt[1, :N, :16], x0,
                  W1_l.T, b1_l[None], W1_r.T, (g1 * s)[None], be1[None])

  agg2 = _seg_sum_call(h1, src_r, dst_r, zeros_feat, width=H)
  wot = jnp.zeros((H, H), jnp.float32).at[:, 0].set(W_out[0])
  bo = jnp.zeros((1, H), jnp.float32).at[0, 0].set(b_out[0])
  out_full = _fin_layer(agg2[0, :N_U], agg2[1, :N_U],
                        cnt[0, :N_U, :16], cnt[1, :N_U, :16],
                        h1[:N_U], x0[:N_U],
                        W2_l.T, b2_l[None], W2_r.T, (g2 * s)[None],
                        be2[None], wot, bo)
  return out_full[:, :1]


# speedup vs baseline: 3.7045x; 1.0713x over previous
import functools

import jax
import jax.numpy as jnp
from jax import lax
from jax.experimental import pallas as pl
from jax.experimental.pallas import tpu as pltpu
from jax.experimental.pallas import tpu_sc as plsc

N_U = 5000
N_P = 5000
N = N_U + N_P
E = 320000
H = 128
EPS = 1e-5
HA = 144

NC = 2
NS = 16
NW = NC * NS
E2 = 2 * E
BE = 128
EP = 655360
NB = EP // (NW * BE)
CB = 16
AGG_ROWS = 10112
ZROWS = AGG_ROWS // NS
ROWS_OUT = ZROWS


def _seg_sum_call(feat, src_r, dst_r, zeros_feat, width):
  out_type = [jax.ShapeDtypeStruct((NC, AGG_ROWS, width), jnp.float32)]
  scratch = [
      pltpu.VMEM((CB, BE), jnp.int32),
      pltpu.VMEM((CB, BE), jnp.int32),
      pltpu.VMEM((BE, width), jnp.float32),
      pltpu.VMEM((BE, width), jnp.float32),
      pltpu.VMEM_SHARED((AGG_ROWS, width), jnp.float32),
      pltpu.SemaphoreType.DMA,
  ]

  mesh = plsc.VectorSubcoreMesh(core_axis_name="c", subcore_axis_name="s",
                                num_cores=NC, num_subcores=NS)

  def body(feat_h, src_h, dst_h, zf_h, out_agg, sidx, didx, rows0, rows1,
           agg, sem):
    c = lax.axis_index("c")
    s = lax.axis_index("s")
    w = c * NS + s
    bufs = (rows0, rows1)

    pltpu.sync_copy(zf_h.at[pl.ds(s * ZROWS, ZROWS)],
                    agg.at[pl.ds(s * ZROWS, ZROWS)])
    plsc.subcore_barrier()

    def chunk(o, carry):
      pltpu.sync_copy(src_h.at[w, pl.ds(o * CB, CB)], sidx)
      pltpu.sync_copy(dst_h.at[w, pl.ds(o * CB, CB)], didx)
      d = pltpu.async_copy(feat_h.at[sidx.at[0]], bufs[0], sem)
      for j in range(CB):
        d.wait()
        if j + 1 < CB:
          d = pltpu.async_copy(feat_h.at[sidx.at[j + 1]], bufs[(j + 1) % 2],
                               sem)
        pltpu.sync_copy(bufs[j % 2], agg.at[didx.at[j]], add=True)
      return carry

    lax.fori_loop(0, NB // CB, chunk, 0)
    plsc.subcore_barrier()

    pltpu.sync_copy(agg.at[pl.ds(s * ROWS_OUT, ROWS_OUT)],
                    out_agg.at[c, pl.ds(s * ROWS_OUT, ROWS_OUT)])

  fn = pl.kernel(body, out_type=out_type, mesh=mesh, scratch_types=scratch)
  return fn(feat, src_r, dst_r, zeros_feat)[0]


def _count_call(dst_r, zeros_feat, ones_rows):
  out_type = [jax.ShapeDtypeStruct((NC, AGG_ROWS, H), jnp.float32)]
  scratch = [
      pltpu.VMEM((CB, BE), jnp.int32),
      pltpu.VMEM((BE, H), jnp.float32),
      pltpu.VMEM_SHARED((AGG_ROWS, H), jnp.float32),
  ]
  mesh = plsc.VectorSubcoreMesh(core_axis_name="c", subcore_axis_name="s",
                                num_cores=NC, num_subcores=NS)

  def body(dst_h, zf_h, ones_h, out_agg, didx, ones, agg):
    c = lax.axis_index("c")
    s = lax.axis_index("s")
    w = c * NS + s
    pltpu.sync_copy(zf_h.at[pl.ds(s * ZROWS, ZROWS)],
                    agg.at[pl.ds(s * ZROWS, ZROWS)])
    pltpu.sync_copy(ones_h, ones)
    plsc.subcore_barrier()

    def chunk(o, carry):
      pltpu.sync_copy(dst_h.at[w, pl.ds(o * CB, CB)], didx)
      for j in range(CB):
        pltpu.sync_copy(ones, agg.at[didx.at[j]], add=True)
      return carry

    lax.fori_loop(0, NB // CB, chunk, 0)
    plsc.subcore_barrier()
    pltpu.sync_copy(agg.at[pl.ds(s * ROWS_OUT, ROWS_OUT)],
                    out_agg.at[c, pl.ds(s * ROWS_OUT, ROWS_OUT)])

  fn = pl.kernel(body, out_type=out_type, mesh=mesh, scratch_types=scratch)
  return fn(dst_r, zeros_feat, ones_rows)[0]


def _enc_body(x_ref, wt_ref, b_ref, o_ref):
  o_ref[...] = jnp.maximum(
      jnp.dot(x_ref[0], wt_ref[0], preferred_element_type=jnp.float32)
      + b_ref[0], 0.0)[None]


def _encode(xs, wts, bs):
  return pl.pallas_call(
      _enc_body,
      grid=(2,),
      in_specs=[
          pl.BlockSpec((1, N_U, H), lambda g: (g, 0, 0)),
          pl.BlockSpec((1, H, H), lambda g: (g, 0, 0)),
          pl.BlockSpec((1, 1, H), lambda g: (g, 0, 0)),
      ],
      out_specs=pl.BlockSpec((1, N_U, H), lambda g: (g, 0, 0)),
      out_shape=jax.ShapeDtypeStruct((2, N_U, H), jnp.float32),
  )(xs, wts, bs)


def _mid_body(aggA, aggB, cA, cB, x_ref, wlt, bl, wrt, gs, be, o_ref):
  cnt = cA[:, :1] + cB[:, :1]
  mean = (aggA[...] + aggB[...]) / jnp.maximum(cnt, 1.0)
  h = (jnp.dot(mean, wlt[...], preferred_element_type=jnp.float32) + bl[...]
       + jnp.dot(x_ref[...], wrt[...], preferred_element_type=jnp.float32))
  o_ref[...] = jnp.maximum(h * gs[...] + be[...], 0.0)


def _mid_layer(aggA, aggB, cA, cB, x, wlt, bl, wrt, gs, be):
  n = x.shape[0]
  args = (aggA, aggB, cA, cB, x, wlt, bl, wrt, gs, be)
  specs = [pl.BlockSpec(a.shape, lambda *_: tuple(0 for _ in a.shape))
           for a in args]
  return pl.pallas_call(
      _mid_body,
      in_specs=specs,
      out_specs=pl.BlockSpec((n, H), lambda *_: (0, 0)),
      out_shape=jax.ShapeDtypeStruct((n, H), jnp.float32),
  )(*args)


def _fin_body(aggA, aggB, cA, cB, h1, x0, wlt, bl, wrt, gs, be, wot, bo,
              o_ref):
  cnt = cA[:, :1] + cB[:, :1]
  mean = (aggA[...] + aggB[...]) / jnp.maximum(cnt, 1.0)
  h = (jnp.dot(mean, wlt[...], preferred_element_type=jnp.float32) + bl[...]
       + jnp.dot(h1[...], wrt[...], preferred_element_type=jnp.float32))
  h = jnp.maximum(h * gs[...] + be[...], 0.0) + x0[...]
  o_ref[...] = jnp.dot(h, wot[...], preferred_element_type=jnp.float32) + bo[...]


def _fin_layer(aggA, aggB, cA, cB, h1, x0, wlt, bl, wrt, gs, be, wot, bo):
  args = (aggA, aggB, cA, cB, h1, x0, wlt, bl, wrt, gs, be, wot, bo)
  specs = [pl.BlockSpec(a.shape, lambda *_: tuple(0 for _ in a.shape))
           for a in args]
  return pl.pallas_call(
      _fin_body,
      in_specs=specs,
      out_specs=pl.BlockSpec((N_U, H), lambda *_: (0, 0)),
      out_shape=jax.ShapeDtypeStruct((N_U, H), jnp.float32),
  )(*args)


def kernel(x_u, x_p, edge_index, W_u, b_u, W_p, b_p, W1_l, b1_l, W1_r, g1,
           be1, W2_l, b2_l, W2_r, g2, be2, W_out, b_out):
  s = 1.0 / jnp.sqrt(jnp.float32(1.0 + EPS))

  src = jnp.concatenate([edge_index[0], edge_index[1]]).astype(jnp.int32)
  dst = jnp.concatenate([edge_index[1], edge_index[0]]).astype(jnp.int32)
  pad = EP - E2
  src_r = jnp.concatenate([src, jnp.zeros((pad,), jnp.int32)]
                          ).reshape(NW, NB, BE)
  dst_r = jnp.concatenate([dst, jnp.full((pad,), N, jnp.int32)]
                          ).reshape(NW, NB, BE)
  zeros_feat = jnp.zeros((AGG_ROWS, H), jnp.float32)
  ones_rows = jnp.zeros((BE, H), jnp.float32).at[:, 0].set(1.0)

  xs = jnp.stack([x_u, x_p])
  wts = jnp.stack([W_u.T, W_p.T])
  bs = jnp.stack([b_u[None], b_p[None]])
  x0 = _encode(xs, wts, bs).reshape(N, H)

  cnt = _count_call(dst_r, zeros_feat, ones_rows)
  tok = (cnt[0, 0, 1] * 0.0).astype(jnp.int32)
  agg1 = _seg_sum_call(x0, src_r + tok, dst_r, zeros_feat, width=H)
  h1 = _mid_layer(agg1[0, :N], agg1[1, :N],
                  cnt[0, :N, :16], cnt[1, :N, :16], x0,
                  W1_l.T, b1_l[None], W1_r.T, (g1 * s)[None], be1[None])

  agg2 = _seg_sum_call(h1, src_r, dst_r, zeros_feat, width=H)
  wot = jnp.zeros((H, H), jnp.float32).at[:, 0].set(W_out[0])
  bo = jnp.zeros((1, H), jnp.float32).at[0, 0].set(b_out[0])
  out_full = _fin_layer(agg2[0, :N_U], agg2[1, :N_U],
                        cnt[0, :N_U, :16], cnt[1, :N_U, :16],
                        h1[:N_U], x0[:N_U],
                        W2_l.T, b2_l[None], W2_r.T, (g2 * s)[None],
                        be2[None], wot, bo)
  return out_full[:, :1]
